# baseline (speedup 1.0000x reference)
import functools

import jax
import jax.numpy as jnp
from jax import lax
from jax.experimental import pallas as pl
from jax.experimental.pallas import tpu as pltpu
from jax.experimental.pallas import tpu_sc as plsc

_N = 8192
_T = 64
_D = 128
_B = 16
_NSC = 4096
_NC = 2
_NS = 16
_NW = _NC * _NS
_RPW = _NSC // _NW
_CH = 64
_NCHUNK = _RPW // _CH
_L = 16
_TCROWS = 512


def _row_body(p, t_v, v_v, w_v, b_v, o_v):
    iota = lax.iota(jnp.int32, _L)
    bidx = [jnp.bitwise_xor(iota, kk) for kk in (1, 2, 4, 8)]

    @plsc.parallel_loop(0, _CH, 1, unroll=8)
    def row(r):
        rt = r * _T
        rd = r * _D
        t0 = t_v[p, pl.ds(rt, _L)]
        t1 = t_v[p, pl.ds(rt + _L, _L)]
        t2 = t_v[p, pl.ds(rt + 2 * _L, _L)]
        t3 = t_v[p, pl.ds(rt + 3 * _L, _L)]
        v0 = v_v[p, pl.ds(rt, _L)]
        v1 = v_v[p, pl.ds(rt + _L, _L)]
        v2 = v_v[p, pl.ds(rt + 2 * _L, _L)]
        v3 = v_v[p, pl.ds(rt + 3 * _L, _L)]
        acc_c = (t0 * v0 + t1 * v1) + (t2 * v2 + t3 * v3)
        acc_s = (t0 + t1) + (t2 + t3)
        for idx in bidx:
            acc_c = acc_c + jnp.take(acc_c, idx)
            acc_s = acc_s + jnp.take(acc_s, idx)
        zero = acc_s == 0.0
        scale = jnp.where(zero, 0.0, acc_c / jnp.where(zero, 1.0, acc_s))
        keep = jnp.where(zero, 0.0, 1.0)
        for j in range(_D // _L):
            w = w_v[p, pl.ds(rd + j * _L, _L)]
            bb = b_v[p, pl.ds(rd + j * _L, _L)]
            o_v[p, pl.ds(rd + j * _L, _L)] = scale * w + keep * bb


_mesh = plsc.VectorSubcoreMesh(core_axis_name="c", subcore_axis_name="s")


@functools.partial(
    pl.kernel,
    mesh=_mesh,
    out_type=jax.ShapeDtypeStruct((_NSC * _D,), jnp.float32),
    scratch_types=[
        pltpu.VMEM((2, _CH * _T), jnp.float32),
        pltpu.VMEM((2, _CH * _T), jnp.float32),
        pltpu.VMEM((2, _CH * _D), jnp.float32),
        pltpu.VMEM((2, _CH * _D), jnp.float32),
        pltpu.VMEM((2, _CH * _D), jnp.float32),
        pltpu.SemaphoreType.DMA,
        pltpu.SemaphoreType.DMA,
        pltpu.SemaphoreType.DMA,
        pltpu.SemaphoreType.DMA,
    ],
    compiler_params=pltpu.CompilerParams(needs_layout_passes=False),
)
def _sc_kernel(t_hbm, v_hbm, w_hbm, b_hbm, o_hbm, t_v, v_v, w_v, b_v, o_v,
               ld_sem0, ld_sem1, st_sem0, st_sem1):
    wid = lax.axis_index("s") * _NC + lax.axis_index("c")
    base = wid * _RPW
    ld_sems = (ld_sem0, ld_sem1)
    st_sems = (st_sem0, st_sem1)

    def issue_loads(g):
        p = g % 2
        r0 = base + g * _CH
        sem = ld_sems[p]
        return [
            pltpu.async_copy(t_hbm.at[pl.ds(r0 * _T, _CH * _T)], t_v.at[p], sem),
            pltpu.async_copy(v_hbm.at[pl.ds(r0 * _T, _CH * _T)], v_v.at[p], sem),
            pltpu.async_copy(w_hbm.at[pl.ds(r0 * _D, _CH * _D)], w_v.at[p], sem),
            pltpu.async_copy(b_hbm.at[pl.ds(r0 * _D, _CH * _D)], b_v.at[p], sem),
        ]

    loads = {0: issue_loads(0)}
    stores = {}
    for g in range(_NCHUNK):
        p = g % 2
        r0 = base + g * _CH
        if g + 1 < _NCHUNK:
            loads[g + 1] = issue_loads(g + 1)
        for h in loads.pop(g):
            h.wait()
        if g >= 2:
            stores.pop(g - 2).wait()
        _row_body(p, t_v, v_v, w_v, b_v, o_v)
        stores[g] = pltpu.async_copy(
            o_v.at[p], o_hbm.at[pl.ds(r0 * _D, _CH * _D)], st_sems[p])
    for g in sorted(stores):
        stores.pop(g).wait()


def _tc_body(t_ref, v_ref, w_ref, b_ref, o_ref):
    t = t_ref[...]
    v = v_ref[...]
    s = jnp.sum(t, axis=1, keepdims=True)
    c = jnp.sum(t * v, axis=1, keepdims=True)
    scale = jnp.where(s == 0.0, 0.0, c / jnp.where(s == 0.0, 1.0, s))
    keep = jnp.where(s == 0.0, 0.0, 1.0)
    o_ref[...] = scale * w_ref[...] + keep * b_ref[...]


def _tc_tail(times, values, W, b):
    ntc = _N - _NSC
    nb = _NSC // _TCROWS
    return pl.pallas_call(
        _tc_body,
        grid=(ntc // _TCROWS,),
        in_specs=[
            pl.BlockSpec((_TCROWS, _T), lambda i: (i + nb, 0)),
            pl.BlockSpec((_TCROWS, _T), lambda i: (i + nb, 0)),
            pl.BlockSpec((_TCROWS, _D), lambda i: (i + nb, 0)),
            pl.BlockSpec((_TCROWS, _D), lambda i: (i + nb, 0)),
        ],
        out_specs=pl.BlockSpec((_TCROWS, _D), lambda i: (i, 0)),
        out_shape=jax.ShapeDtypeStruct((ntc, _D), jnp.float32),
    )(times, values, W, b)


def kernel(measurement_times, measurement_values, W, b):
    o_sc = _sc_kernel(measurement_times.reshape(-1),
                      measurement_values.reshape(-1),
                      W.reshape(-1), b.reshape(-1))
    o_tc = _tc_tail(measurement_times, measurement_values, W, b)
    out = jnp.concatenate([o_sc.reshape(_NSC, _D), o_tc], axis=0)
    return out.reshape(_B, _N // _B, _D)


# speedup vs baseline: 1.1486x; 1.1486x over previous
import functools

import jax
import jax.numpy as jnp
from jax import lax
from jax.experimental import pallas as pl
from jax.experimental.pallas import tpu as pltpu
from jax.experimental.pallas import tpu_sc as plsc

_N = 8192
_T = 64
_D = 128
_B = 16
_NC = 2
_NS = 16
_NW = _NC * _NS
_RPW = _N // _NW
_CH = 64
_NCHUNK = _RPW // _CH
_L = 16


def _row_body(p, t_v, v_v, w_v, b_v, o_v):
    iota = lax.iota(jnp.int32, _L)
    bidx = [jnp.bitwise_xor(iota, kk) for kk in (1, 2, 4, 8)]

    @plsc.parallel_loop(0, _CH, 1, unroll=8)
    def row(r):
        rt = r * _T
        rd = r * _D
        t0 = t_v[p, pl.ds(rt, _L)]
        t1 = t_v[p, pl.ds(rt + _L, _L)]
        t2 = t_v[p, pl.ds(rt + 2 * _L, _L)]
        t3 = t_v[p, pl.ds(rt + 3 * _L, _L)]
        v0 = v_v[p, pl.ds(rt, _L)]
        v1 = v_v[p, pl.ds(rt + _L, _L)]
        v2 = v_v[p, pl.ds(rt + 2 * _L, _L)]
        v3 = v_v[p, pl.ds(rt + 3 * _L, _L)]
        acc_c = (t0 * v0 + t1 * v1) + (t2 * v2 + t3 * v3)
        acc_s = (t0 + t1) + (t2 + t3)
        for idx in bidx:
            acc_c = acc_c + jnp.take(acc_c, idx)
            acc_s = acc_s + jnp.take(acc_s, idx)
        zero = acc_s == 0.0
        sdiv = jnp.where(zero, 1.0, acc_s)
        xb = lax.bitcast_convert_type(sdiv, jnp.int32)
        x = lax.bitcast_convert_type(jnp.int32(0x7EF311C3) - xb, jnp.float32)
        x = x * (2.0 - sdiv * x)
        x = x * (2.0 - sdiv * x)
        x = x * (2.0 - sdiv * x)
        scale = jnp.where(zero, 0.0, acc_c * x)
        keep = jnp.where(zero, 0.0, 1.0)
        for j in range(_D // _L):
            w = w_v[p, pl.ds(rd + j * _L, _L)]
            bb = b_v[p, pl.ds(rd + j * _L, _L)]
            o_v[p, pl.ds(rd + j * _L, _L)] = scale * w + keep * bb


_mesh = plsc.VectorSubcoreMesh(core_axis_name="c", subcore_axis_name="s")


@functools.partial(
    pl.kernel,
    mesh=_mesh,
    out_type=jax.ShapeDtypeStruct((_N * _D,), jnp.float32),
    scratch_types=[
        pltpu.VMEM((2, _CH * _T), jnp.float32),
        pltpu.VMEM((2, _CH * _T), jnp.float32),
        pltpu.VMEM((2, _CH * _D), jnp.float32),
        pltpu.VMEM((2, _CH * _D), jnp.float32),
        pltpu.VMEM((2, _CH * _D), jnp.float32),
        pltpu.SemaphoreType.DMA,
        pltpu.SemaphoreType.DMA,
        pltpu.SemaphoreType.DMA,
        pltpu.SemaphoreType.DMA,
    ],
    compiler_params=pltpu.CompilerParams(needs_layout_passes=False),
)
def _sc_kernel(t_hbm, v_hbm, w_hbm, b_hbm, o_hbm, t_v, v_v, w_v, b_v, o_v,
               ld_sem0, ld_sem1, st_sem0, st_sem1):
    wid = lax.axis_index("s") * _NC + lax.axis_index("c")
    base = wid * _RPW
    ld_sems = (ld_sem0, ld_sem1)
    st_sems = (st_sem0, st_sem1)

    def issue_loads(g):
        p = g % 2
        r0 = base + g * _CH
        sem = ld_sems[p]
        return [
            pltpu.async_copy(t_hbm.at[pl.ds(r0 * _T, _CH * _T)], t_v.at[p], sem),
            pltpu.async_copy(v_hbm.at[pl.ds(r0 * _T, _CH * _T)], v_v.at[p], sem),
            pltpu.async_copy(w_hbm.at[pl.ds(r0 * _D, _CH * _D)], w_v.at[p], sem),
            pltpu.async_copy(b_hbm.at[pl.ds(r0 * _D, _CH * _D)], b_v.at[p], sem),
        ]

    loads = {0: issue_loads(0)}
    stores = {}
    for g in range(_NCHUNK):
        p = g % 2
        r0 = base + g * _CH
        if g + 1 < _NCHUNK:
            loads[g + 1] = issue_loads(g + 1)
        for h in loads.pop(g):
            h.wait()
        if g >= 2:
            stores.pop(g - 2).wait()
        _row_body(p, t_v, v_v, w_v, b_v, o_v)
        stores[g] = pltpu.async_copy(
            o_v.at[p], o_hbm.at[pl.ds(r0 * _D, _CH * _D)], st_sems[p])
    for g in sorted(stores):
        stores.pop(g).wait()


def kernel(measurement_times, measurement_values, W, b):
    out = _sc_kernel(measurement_times.reshape(-1), measurement_values.reshape(-1),
                     W.reshape(-1), b.reshape(-1))
    return out.reshape(_B, _N // _B, _D)


# speedup vs baseline: 1.1836x; 1.0305x over previous
import functools

import jax
import jax.numpy as jnp
from jax import lax
from jax.experimental import pallas as pl
from jax.experimental.pallas import tpu as pltpu
from jax.experimental.pallas import tpu_sc as plsc

_N = 8192
_T = 64
_D = 128
_B = 16
_NC = 2
_NS = 16
_NW = _NC * _NS
_RPW = _N // _NW
_CH = 64
_NCHUNK = _RPW // _CH
_L = 16


def _row_body(p, t_v, v_v, w_v, b_v, o_v, sc_v, kp_v):
    iota = lax.iota(jnp.int32, _L)

    @plsc.parallel_loop(0, _CH, 1, unroll=4)
    def scales(r):
        t0 = t_v[p, r, pl.ds(0, _L)]
        t1 = t_v[p, r, pl.ds(_L, _L)]
        t2 = t_v[p, r, pl.ds(2 * _L, _L)]
        t3 = t_v[p, r, pl.ds(3 * _L, _L)]
        v0 = v_v[p, r, pl.ds(0, _L)]
        v1 = v_v[p, r, pl.ds(_L, _L)]
        v2 = v_v[p, r, pl.ds(2 * _L, _L)]
        v3 = v_v[p, r, pl.ds(3 * _L, _L)]
        acc_c = (t0 * v0 + t1 * v1) + (t2 * v2 + t3 * v3)
        acc_s = (t0 + t1) + (t2 + t3)
        for kk in (1, 2, 4, 8):
            idx = jnp.bitwise_xor(iota, kk)
            acc_c = acc_c + jnp.take(acc_c, idx)
            acc_s = acc_s + jnp.take(acc_s, idx)
        zero = acc_s == 0.0
        scale = jnp.where(zero, 0.0, acc_c / jnp.where(zero, 1.0, acc_s))
        keep = jnp.where(zero, 0.0, 1.0)
        sc_v[r, :] = scale
        kp_v[r, :] = keep

    @plsc.parallel_loop(0, _CH, 1, unroll=8)
    def row(r):
        scale = sc_v[r, :]
        keep = kp_v[r, :]
        for j in range(_D // _L):
            w = w_v[p, r, pl.ds(j * _L, _L)]
            bb = b_v[p, r, pl.ds(j * _L, _L)]
            o_v[p, r, pl.ds(j * _L, _L)] = scale * w + keep * bb


_mesh = plsc.VectorSubcoreMesh(core_axis_name="c", subcore_axis_name="s")


@functools.partial(
    pl.kernel,
    mesh=_mesh,
    out_type=jax.ShapeDtypeStruct((_N, _D), jnp.float32),
    scratch_types=[
        pltpu.VMEM((2, _CH, _T), jnp.float32),
        pltpu.VMEM((2, _CH, _T), jnp.float32),
        pltpu.VMEM((2, _CH, _D), jnp.float32),
        pltpu.VMEM((2, _CH, _D), jnp.float32),
        pltpu.VMEM((2, _CH, _D), jnp.float32),
        pltpu.VMEM((_CH, _L), jnp.float32),
        pltpu.VMEM((_CH, _L), jnp.float32),
        pltpu.SemaphoreType.DMA,
        pltpu.SemaphoreType.DMA,
        pltpu.SemaphoreType.DMA,
        pltpu.SemaphoreType.DMA,
    ],
    compiler_params=pltpu.CompilerParams(needs_layout_passes=False),
)
def _sc_kernel(t_hbm, v_hbm, w_hbm, b_hbm, o_hbm, t_v, v_v, w_v, b_v, o_v,
               sc_v, kp_v, ld_sem0, ld_sem1, st_sem0, st_sem1):
    wid = lax.axis_index("s") * _NC + lax.axis_index("c")
    base = wid * _RPW
    ld_sems = (ld_sem0, ld_sem1)
    st_sems = (st_sem0, st_sem1)

    def issue_loads(g):
        p = g % 2
        r0 = base + g * _CH
        sem = ld_sems[p]
        return [
            pltpu.async_copy(t_hbm.at[pl.ds(r0, _CH), :], t_v.at[p], sem),
            pltpu.async_copy(v_hbm.at[pl.ds(r0, _CH), :], v_v.at[p], sem),
            pltpu.async_copy(w_hbm.at[pl.ds(r0, _CH), :], w_v.at[p], sem),
            pltpu.async_copy(b_hbm.at[pl.ds(r0, _CH), :], b_v.at[p], sem),
        ]

    loads = {0: issue_loads(0)}
    stores = {}
    for g in range(_NCHUNK):
        p = g % 2
        r0 = base + g * _CH
        if g + 1 < _NCHUNK:
            loads[g + 1] = issue_loads(g + 1)
        for h in loads.pop(g):
            h.wait()
        if g >= 2:
            stores.pop(g - 2).wait()
        _row_body(p, t_v, v_v, w_v, b_v, o_v, sc_v, kp_v)
        stores[g] = pltpu.async_copy(
            o_v.at[p], o_hbm.at[pl.ds(r0, _CH), :], st_sems[p])
    for g in sorted(stores):
        stores.pop(g).wait()


def kernel(measurement_times, measurement_values, W, b):
    out = _sc_kernel(measurement_times, measurement_values, W, b)
    return out.reshape(_B, _N // _B, _D)
